# Initial kernel scaffold; baseline (speedup 1.0000x reference)
import jax
import jax.numpy as jnp
from jax.experimental import pallas as pl


def kernel(h_node, pos_node, h_bond, bond_index, batch, is_mol, is_frag, params):
    raise NotImplementedError("write your pallas kernel here")


# speedup vs baseline: 2.1699x; 2.1699x over previous
import functools
import jax
import jax.numpy as jnp
from jax import lax
from jax.experimental import pallas as pl
from jax.experimental.pallas import tpu as pltpu
from jax.experimental.pallas import tpu_sc as plsc

N_NODES = 10000
N_EDGES = 160000
D = 128
NUM_GAUSS = 20
CUTOFF = 10.0
F32 = jnp.float32

NC = 2
NS = 16
NW = NC * NS
CHUNK = 128
N_CHUNKS = N_EDGES // CHUNK
CH_PER_W = -(-N_CHUNKS // NW)
RO_CHUNK = 80
N_RO = N_NODES // RO_CHUNK
RO_PER_SUB = -(-N_RO // NS)


def _sc_mesh():
    return plsc.VectorSubcoreMesh(core_axis_name="c", subcore_axis_name="s")


def _worker_id():
    return lax.axis_index("s") * NC + lax.axis_index("c")


def _multi_gather(pairs, tc_tiling=True):
    n = len(pairs)
    widths = [int(t.shape[1]) for t, _ in pairs]
    out_type = [jax.ShapeDtypeStruct((N_EDGES, w), F32) for w in widths]
    scratch = []
    for w in widths:
        scratch.append(pltpu.VMEM((CHUNK,), jnp.int32))
        scratch.append(pltpu.VMEM((CHUNK, w), F32))
    scratch.append(pltpu.SemaphoreType.DMA)

    @functools.partial(pl.kernel, out_type=out_type, mesh=_sc_mesh(),
                       scratch_types=scratch,
                       compiler_params=pltpu.CompilerParams(
                           use_tc_tiling_on_sc=tc_tiling))
    def k(*refs):
        tabs = refs[0:n]
        idxs = refs[n:2 * n]
        outs = refs[2 * n:3 * n]
        sem = refs[-1]
        bufs = refs[3 * n:-1]
        wid = _worker_id()

        def body(i, _):
            c = wid + i * NW

            @pl.when(c < N_CHUNKS)
            def _():
                base = c * CHUNK
                for j in range(n):
                    idx_v = bufs[2 * j]
                    rows_v = bufs[2 * j + 1]
                    pltpu.sync_copy(idxs[j].at[pl.ds(base, CHUNK)], idx_v)
                    pltpu.async_copy(tabs[j].at[idx_v], rows_v, sem).wait()
                    pltpu.sync_copy(rows_v, outs[j].at[pl.ds(base, CHUNK)])
            return 0

        lax.fori_loop(0, CH_PER_W, body, 0, unroll=False)

    res = k(*[t for t, _ in pairs], *[i for _, i in pairs])
    return res if isinstance(res, (list, tuple)) else [res]


def _scatter_add(rows, idx):
    @functools.partial(
        pl.kernel,
        out_type=jax.ShapeDtypeStruct((NC, N_NODES, D), F32),
        mesh=_sc_mesh(),
        scratch_types=[
            pltpu.VMEM((CHUNK,), jnp.int32),
            pltpu.VMEM((CHUNK, D), F32),
            pltpu.VMEM((RO_CHUNK, D), F32),
            pltpu.VMEM_SHARED((N_NODES, D), F32),
        ],
    )
    def k(rows_hbm, idx_hbm, out_hbm, idx_v, rows_v, stage_v, accum_sh):
        cid = lax.axis_index("c")
        sid = lax.axis_index("s")
        wid = sid * NC + cid

        def zrow(r, _):
            def zlane(kk, _):
                stage_v[r, pl.ds(kk * 16, 16)] = jnp.zeros((16,), F32)
                return 0
            lax.fori_loop(0, D // 16, zlane, 0, unroll=False)
            return 0
        lax.fori_loop(0, RO_CHUNK, zrow, 0, unroll=False)

        def zcopy(j, _):
            c = sid + j * NS

            @pl.when(c < N_RO)
            def _():
                pltpu.sync_copy(stage_v, accum_sh.at[pl.ds(c * RO_CHUNK, RO_CHUNK)])
            return 0
        lax.fori_loop(0, RO_PER_SUB, zcopy, 0, unroll=False)
        plsc.subcore_barrier()

        def body(i, _):
            c = wid + i * NW

            @pl.when(c < N_CHUNKS)
            def _():
                base = c * CHUNK
                pltpu.sync_copy(idx_hbm.at[pl.ds(base, CHUNK)], idx_v)
                pltpu.sync_copy(rows_hbm.at[pl.ds(base, CHUNK)], rows_v)
                pltpu.sync_copy(rows_v, accum_sh.at[idx_v], add=True)
            return 0

        lax.fori_loop(0, CH_PER_W, body, 0, unroll=False)
        plsc.subcore_barrier()

        def rocopy(j, _):
            c = sid + j * NS

            @pl.when(c < N_RO)
            def _():
                r0 = c * RO_CHUNK
                pltpu.sync_copy(accum_sh.at[pl.ds(r0, RO_CHUNK)], stage_v)
                pltpu.sync_copy(stage_v, out_hbm.at[cid, pl.ds(r0, RO_CHUNK)])
            return 0
        lax.fori_loop(0, RO_PER_SUB, rocopy, 0, unroll=False)

    return k(rows, idx)


T_N = 2000
T_E = 2000


def _dot(a, b):
    return jnp.dot(a, b, preferred_element_type=F32)


def _ln(x, g, b):
    m = jnp.mean(x, -1, keepdims=True)
    v = jnp.mean((x - m) * (x - m), -1, keepdims=True)
    return (x - m) / jnp.sqrt(v + 1e-5) * g + b


def _row_spec(t, w):
    return pl.BlockSpec((t, w), lambda i: (i, 0))


def _full_spec(shape):
    nd = len(shape)
    return pl.BlockSpec(shape, lambda i: (0,) * nd)


def _tc_call(body, n_rows, tile, ins, outs_w):
    grid = (n_rows // tile,)
    in_specs = [
        _row_spec(tile, int(a.shape[-1])) if tiled else _full_spec(a.shape)
        for a, tiled in ins
    ]
    out_specs = [_row_spec(tile, w) for w in outs_w]
    out_shape = [jax.ShapeDtypeStruct((n_rows, w), F32) for w in outs_w]
    return pl.pallas_call(
        body, grid=grid, in_specs=in_specs, out_specs=out_specs,
        out_shape=out_shape,
    )(*[a for a, _ in ins])


def _r2(x):
    return jnp.reshape(x, (1, -1))


def _node_a_body(x_ref, Wl, bl, W1, b1, W2, b2, Wc, bc,
                 h_ref, hn1_ref, cen1_ref):
    x = x_ref[...]
    h = _dot(x, Wl[...]) + bl[...]
    h_ref[...] = h
    t = jnp.maximum(_dot(h, W1[...]) + b1[...], 0.0)
    hn1_ref[...] = _dot(t, W2[...]) + b2[...]
    cen1_ref[...] = _dot(h, Wc[...]) + bc[...]


def _edge1_body(pr_ref, pc_ref, g1_ref, off_ref, Wemb, bemb, W1, b1, W2, b2,
                Wm, bm, msg_ref):
    v = pr_ref[...] - pc_ref[...]
    d = jnp.sqrt(jnp.sum(v * v, -1, keepdims=True) + 1e-8)
    step = CUTOFF / (NUM_GAUSS - 1)
    coeff = -0.5 / (step * step)
    gs = jnp.exp(coeff * (d - off_ref[...]) ** 2)
    ea = _dot(gs, Wemb[...]) + bemb[...]
    t = jnp.maximum(_dot(ea, W1[...]) + b1[...], 0.0)
    he = _dot(t, W2[...]) + b2[...]
    msg_ref[...] = _dot(he * g1_ref[...], Wm[...]) + bm[...]


def _node_b_body(h_ref, p0_ref, p1_ref, cen_ref, Wo, bo, g, b,
                 WnlL, WnfL, bnfL, WnlR, WnfR, bnfR,
                 W1, b1, W2, b2, Wc2, bc2,
                 h1_ref, catL_ref, catR_ref, hn2_ref, cen2_ref):
    t = cen_ref[...] + p0_ref[...] + p1_ref[...]
    t = _ln(t, g[...], b[...])
    h1 = h_ref[...] + _dot(jnp.maximum(t, 0.0), Wo[...]) + bo[...]
    h1_ref[...] = h1
    catL_ref[:, 0:256] = _dot(h1, WnlL[...])
    catL_ref[:, 256:384] = _dot(h1, WnfL[...]) + bnfL[...]
    catR_ref[:, 0:256] = _dot(h1, WnlR[...])
    catR_ref[:, 256:384] = _dot(h1, WnfR[...]) + bnfR[...]
    u = jnp.maximum(_dot(h1, W1[...]) + b1[...], 0.0)
    hn2_ref[...] = _dot(u, W2[...]) + b2[...]
    cen2_ref[...] = _dot(h1, Wc2[...]) + bc2[...]


def _ebond_body(hb_ref, gl_ref, gr_ref,
                WblL, Wl1, bl1, Wl2, bl2, WblR, Wr1, br1, Wr2, br2,
                ml_ref, mr_ref):
    hb = hb_ref[...]
    gl = gl_ref[...]
    gr = gr_ref[...]
    il = _dot(hb, WblL[...]) * gl[:, 0:256]
    tl = jnp.maximum(_dot(il, Wl1[...]) + bl1[...], 0.0)
    ml_ref[...] = _dot(tl, Wl2[...]) + bl2[...]
    ir = _dot(hb, WblR[...]) * gr[:, 0:256]
    tr = jnp.maximum(_dot(ir, Wr1[...]) + br1[...], 0.0)
    mr_ref[...] = _dot(tr, Wr2[...]) + br2[...]


def _combine_body(a0_ref, a1_ref, b0_ref, b1_ref, a_ref, b_ref):
    a_ref[...] = a0_ref[...] + a1_ref[...]
    b_ref[...] = b0_ref[...] + b1_ref[...]


def _bfin_body(hb_ref, al_ref, ar_ref, gl_ref, gr_ref, g2_ref,
               Ws, bs, g, b, Wo, bo, We1, be1, We2, be2, Wm2, bm2,
               hb1_ref, msg2_ref):
    hb = hb_ref[...]
    s = (al_ref[...] + ar_ref[...] + gl_ref[:, 256:384] + gr_ref[:, 256:384]
         + _dot(hb, Ws[...]) + bs[...])
    s = _ln(s, g[...], b[...])
    hb1 = hb + _dot(jnp.maximum(s, 0.0), Wo[...]) + bo[...]
    hb1_ref[...] = hb1
    t = jnp.maximum(_dot(hb1, We1[...]) + be1[...], 0.0)
    he = _dot(t, We2[...]) + be2[...]
    msg2_ref[...] = _dot(he * g2_ref[...], Wm2[...]) + bm2[...]


def _node_c_body(h1_ref, cen2_ref, q0_ref, q1_ref, Wo, bo, g, b, h2_ref):
    t = cen2_ref[...] + q0_ref[...] + q1_ref[...]
    t = _ln(t, g[...], b[...])
    h2_ref[...] = h1_ref[...] + _dot(jnp.maximum(t, 0.0), Wo[...]) + bo[...]


def kernel(h_node, pos_node, h_bond, bond_index, batch, is_mol, is_frag, params):
    p = params
    nbe, bb, nbb = p["nbe"][0], p["bb"][0], p["nbb"][0]
    row = bond_index[0]
    col = bond_index[1]

    pos16 = jnp.zeros((N_NODES, 16), F32).at[:, 0:3].set(pos_node)
    off = _r2(jnp.linspace(0.0, CUTOFF, NUM_GAUSS).astype(F32))

    h, hn1, cen1 = _tc_call(
        _node_a_body, N_NODES, T_N,
        [(h_node, True),
         (p["lin_node"]["W"], False), (_r2(p["lin_node"]["b"]), False),
         (nbe["node_net"]["l1"]["W"], False), (_r2(nbe["node_net"]["l1"]["b"]), False),
         (nbe["node_net"]["l2"]["W"], False), (_r2(nbe["node_net"]["l2"]["b"]), False),
         (nbe["centroid"]["W"], False), (_r2(nbe["centroid"]["b"]), False)],
        [D, D, D])

    pr, pc = _multi_gather([(pos16, row), (pos16, col)], tc_tiling=False)
    (g1,) = _multi_gather([(hn1, col)])

    (msg,) = _tc_call(
        _edge1_body, N_EDGES, T_E,
        [(pr, True), (pc, True), (g1, True), (off, False),
         (p["edge_emb"]["W"], False), (_r2(p["edge_emb"]["b"]), False),
         (nbe["edge_net"]["l1"]["W"], False), (_r2(nbe["edge_net"]["l1"]["b"]), False),
         (nbe["edge_net"]["l2"]["W"], False), (_r2(nbe["edge_net"]["l2"]["b"]), False),
         (nbe["msg_net"]["W"], False), (_r2(nbe["msg_net"]["b"]), False)],
        [D])

    aggr1p = _scatter_add(msg, row)

    h1, catL, catR, hn2, cen2 = _tc_call(
        _node_b_body, N_NODES, T_N,
        [(h, True), (aggr1p[0], True), (aggr1p[1], True), (cen1, True),
         (nbe["out"]["W"], False), (_r2(nbe["out"]["b"]), False),
         (_r2(nbe["ln_g"]), False), (_r2(nbe["ln_b"]), False),
         (bb["ffn_l"]["node_lin"]["W"], False),
         (bb["node_ffn_l"]["W"], False), (_r2(bb["node_ffn_l"]["b"]), False),
         (bb["ffn_r"]["node_lin"]["W"], False),
         (bb["node_ffn_r"]["W"], False), (_r2(bb["node_ffn_r"]["b"]), False),
         (nbb["node_net"]["l1"]["W"], False), (_r2(nbb["node_net"]["l1"]["b"]), False),
         (nbb["node_net"]["l2"]["W"], False), (_r2(nbb["node_net"]["l2"]["b"]), False),
         (nbb["centroid"]["W"], False), (_r2(nbb["centroid"]["b"]), False)],
        [D, 384, 384, D, D])

    gl, gr, g2 = _multi_gather([(catL, row), (catR, col), (hn2, col)])

    m_l, m_r = _tc_call(
        _ebond_body, N_EDGES, T_E,
        [(h_bond, True), (gl, True), (gr, True),
         (bb["ffn_l"]["bond_lin"]["W"], False),
         (bb["ffn_l"]["inter"]["l1"]["W"], False), (_r2(bb["ffn_l"]["inter"]["l1"]["b"]), False),
         (bb["ffn_l"]["inter"]["l2"]["W"], False), (_r2(bb["ffn_l"]["inter"]["l2"]["b"]), False),
         (bb["ffn_r"]["bond_lin"]["W"], False),
         (bb["ffn_r"]["inter"]["l1"]["W"], False), (_r2(bb["ffn_r"]["inter"]["l1"]["b"]), False),
         (bb["ffn_r"]["inter"]["l2"]["W"], False), (_r2(bb["ffn_r"]["inter"]["l2"]["b"]), False)],
        [D, D])

    aggLp = _scatter_add(m_l, col)
    aggRp = _scatter_add(m_r, row)

    aggL, aggR = _tc_call(
        _combine_body, N_NODES, T_N,
        [(aggLp[0], True), (aggLp[1], True), (aggRp[0], True), (aggRp[1], True)],
        [D, D])

    al, ar = _multi_gather([(aggL, row), (aggR, col)])

    hb1, msg2 = _tc_call(
        _bfin_body, N_EDGES, T_E,
        [(h_bond, True), (al, True), (ar, True), (gl, True), (gr, True), (g2, True),
         (bb["self_ffn"]["W"], False), (_r2(bb["self_ffn"]["b"]), False),
         (_r2(bb["ln_g"]), False), (_r2(bb["ln_b"]), False),
         (bb["out"]["W"], False), (_r2(bb["out"]["b"]), False),
         (nbb["edge_net"]["l1"]["W"], False), (_r2(nbb["edge_net"]["l1"]["b"]), False),
         (nbb["edge_net"]["l2"]["W"], False), (_r2(nbb["edge_net"]["l2"]["b"]), False),
         (nbb["msg_net"]["W"], False), (_r2(nbb["msg_net"]["b"]), False)],
        [D, D])

    aggr2p = _scatter_add(msg2, row)

    (h2,) = _tc_call(
        _node_c_body, N_NODES, T_N,
        [(h1, True), (cen2, True), (aggr2p[0], True), (aggr2p[1], True),
         (nbb["out"]["W"], False), (_r2(nbb["out"]["b"]), False),
         (_r2(nbb["ln_g"]), False), (_r2(nbb["ln_b"]), False)],
        [D])

    return h2, hb1


# speedup vs baseline: 2.5779x; 1.1880x over previous
import functools
import jax
import jax.numpy as jnp
from jax import lax
from jax.experimental import pallas as pl
from jax.experimental.pallas import tpu as pltpu
from jax.experimental.pallas import tpu_sc as plsc

N_NODES = 10000
N_EDGES = 160000
D = 128
NUM_GAUSS = 20
CUTOFF = 10.0
F32 = jnp.float32

NC = 2
NS = 16
NW = NC * NS
CHUNK = 128
N_CHUNKS = N_EDGES // CHUNK
CH_PER_W = -(-N_CHUNKS // NW)
RO_CHUNK = 80
N_RO = N_NODES // RO_CHUNK
RO_PER_SUB = -(-N_RO // NS)


def _sc_mesh():
    return plsc.VectorSubcoreMesh(core_axis_name="c", subcore_axis_name="s")


def _worker_id():
    return lax.axis_index("s") * NC + lax.axis_index("c")


def _multi_gather(pairs, tc_tiling=True):
    n = len(pairs)
    widths = [int(t.shape[1]) for t, _ in pairs]
    out_type = [jax.ShapeDtypeStruct((N_EDGES, w), F32) for w in widths]
    scratch = []
    for w in widths:
        scratch.append(pltpu.VMEM((CHUNK,), jnp.int32))
        scratch.append(pltpu.VMEM((CHUNK, w), F32))
        scratch.append(pltpu.SemaphoreType.DMA)

    @functools.partial(pl.kernel, out_type=out_type, mesh=_sc_mesh(),
                       scratch_types=scratch,
                       compiler_params=pltpu.CompilerParams(
                           use_tc_tiling_on_sc=tc_tiling))
    def k(*refs):
        tabs = refs[0:n]
        idxs = refs[n:2 * n]
        outs = refs[2 * n:3 * n]
        bufs = refs[3 * n:]
        wid = _worker_id()

        def body(i, _):
            c = wid + i * NW

            @pl.when(c < N_CHUNKS)
            def _():
                base = c * CHUNK
                di = [pltpu.async_copy(idxs[j].at[pl.ds(base, CHUNK)],
                                       bufs[3 * j], bufs[3 * j + 2])
                      for j in range(n)]
                dg = []
                for j in range(n):
                    di[j].wait()
                    dg.append(pltpu.async_copy(tabs[j].at[bufs[3 * j]],
                                               bufs[3 * j + 1], bufs[3 * j + 2]))
                ds = []
                for j in range(n):
                    dg[j].wait()
                    ds.append(pltpu.async_copy(bufs[3 * j + 1],
                                               outs[j].at[pl.ds(base, CHUNK)],
                                               bufs[3 * j + 2]))
                for j in range(n):
                    ds[j].wait()
            return 0

        lax.fori_loop(0, CH_PER_W, body, 0, unroll=False)

    res = k(*[t for t, _ in pairs], *[i for _, i in pairs])
    return res if isinstance(res, (list, tuple)) else [res]


def _scatter_add(rows, idx):
    @functools.partial(
        pl.kernel,
        out_type=jax.ShapeDtypeStruct((NC, N_NODES, D), F32),
        mesh=_sc_mesh(),
        scratch_types=[
            pltpu.VMEM((CHUNK,), jnp.int32),
            pltpu.VMEM((CHUNK, D), F32),
            pltpu.VMEM((CHUNK,), jnp.int32),
            pltpu.VMEM((CHUNK, D), F32),
            pltpu.SemaphoreType.DMA,
            pltpu.SemaphoreType.DMA,
            pltpu.VMEM((RO_CHUNK, D), F32),
            pltpu.VMEM_SHARED((N_NODES, D), F32),
        ],
    )
    def k(rows_hbm, idx_hbm, out_hbm, idx_v0, rows_v0, idx_v1, rows_v1,
          sem0, sem1, stage_v, accum_sh):
        cid = lax.axis_index("c")
        sid = lax.axis_index("s")
        wid = sid * NC + cid

        def zrow(r, _):
            def zlane(kk, _):
                stage_v[r, pl.ds(kk * 16, 16)] = jnp.zeros((16,), F32)
                return 0
            lax.fori_loop(0, D // 16, zlane, 0, unroll=False)
            return 0
        lax.fori_loop(0, RO_CHUNK, zrow, 0, unroll=False)

        def zcopy(j, _):
            c = sid + j * NS

            @pl.when(c < N_RO)
            def _():
                pltpu.sync_copy(stage_v, accum_sh.at[pl.ds(c * RO_CHUNK, RO_CHUNK)])
            return 0
        lax.fori_loop(0, RO_PER_SUB, zcopy, 0, unroll=False)
        plsc.subcore_barrier()

        slots = [(idx_v0, rows_v0, sem0), (idx_v1, rows_v1, sem1)]

        def body(i2, _):
            for b in range(2):
                c = wid + (2 * i2 + b) * NW

                @pl.when(c < N_CHUNKS)
                def _(b=b, c=c):
                    iv, rv, sm = slots[b]
                    base = c * CHUNK
                    pltpu.async_copy(idx_hbm.at[pl.ds(base, CHUNK)], iv, sm)
                    pltpu.async_copy(rows_hbm.at[pl.ds(base, CHUNK)], rv, sm)
            for b in range(2):
                c = wid + (2 * i2 + b) * NW

                @pl.when(c < N_CHUNKS)
                def _(b=b, c=c):
                    iv, rv, sm = slots[b]
                    base = c * CHUNK
                    pltpu.make_async_copy(idx_hbm.at[pl.ds(base, CHUNK)], iv, sm).wait()
                    pltpu.make_async_copy(rows_hbm.at[pl.ds(base, CHUNK)], rv, sm).wait()
                    pltpu.sync_copy(rv, accum_sh.at[iv], add=True)
            return 0

        lax.fori_loop(0, -(-CH_PER_W // 2), body, 0, unroll=False)
        plsc.subcore_barrier()

        def rocopy(j, _):
            c = sid + j * NS

            @pl.when(c < N_RO)
            def _():
                r0 = c * RO_CHUNK
                pltpu.sync_copy(accum_sh.at[pl.ds(r0, RO_CHUNK)], stage_v)
                pltpu.sync_copy(stage_v, out_hbm.at[cid, pl.ds(r0, RO_CHUNK)])
            return 0
        lax.fori_loop(0, RO_PER_SUB, rocopy, 0, unroll=False)

    return k(rows, idx)


T_N = 2000
T_E = 2000


def _dot(a, b):
    return jnp.dot(a, b, preferred_element_type=F32)


def _ln(x, g, b):
    m = jnp.mean(x, -1, keepdims=True)
    v = jnp.mean((x - m) * (x - m), -1, keepdims=True)
    return (x - m) / jnp.sqrt(v + 1e-5) * g + b


def _row_spec(t, w):
    return pl.BlockSpec((t, w), lambda i: (i, 0))


def _full_spec(shape):
    nd = len(shape)
    return pl.BlockSpec(shape, lambda i: (0,) * nd)


def _tc_call(body, n_rows, tile, ins, outs_w):
    grid = (n_rows // tile,)
    in_specs = [
        _row_spec(tile, int(a.shape[-1])) if tiled else _full_spec(a.shape)
        for a, tiled in ins
    ]
    out_specs = [_row_spec(tile, w) for w in outs_w]
    out_shape = [jax.ShapeDtypeStruct((n_rows, w), F32) for w in outs_w]
    return pl.pallas_call(
        body, grid=grid, in_specs=in_specs, out_specs=out_specs,
        out_shape=out_shape,
    )(*[a for a, _ in ins])


def _r2(x):
    return jnp.reshape(x, (1, -1))


def _node_a_body(x_ref, Wl, bl, W1, b1, W2, b2, Wc, bc,
                 h_ref, hn1_ref, cen1_ref):
    x = x_ref[...]
    h = _dot(x, Wl[...]) + bl[...]
    h_ref[...] = h
    t = jnp.maximum(_dot(h, W1[...]) + b1[...], 0.0)
    hn1_ref[...] = _dot(t, W2[...]) + b2[...]
    cen1_ref[...] = _dot(h, Wc[...]) + bc[...]


def _edge1_body(pr_ref, pc_ref, g1_ref, off_ref, Wemb, bemb, W1, b1, W2, b2,
                Wm, bm, msg_ref):
    v = pr_ref[...] - pc_ref[...]
    d = jnp.sqrt(jnp.sum(v * v, -1, keepdims=True) + 1e-8)
    step = CUTOFF / (NUM_GAUSS - 1)
    coeff = -0.5 / (step * step)
    gs = jnp.exp(coeff * (d - off_ref[...]) ** 2)
    ea = _dot(gs, Wemb[...]) + bemb[...]
    t = jnp.maximum(_dot(ea, W1[...]) + b1[...], 0.0)
    he = _dot(t, W2[...]) + b2[...]
    msg_ref[...] = _dot(he * g1_ref[...], Wm[...]) + bm[...]


def _node_b_body(h_ref, p0_ref, p1_ref, cen_ref, Wo, bo, g, b,
                 WnlL, WnfL, bnfL, WnlR, WnfR, bnfR,
                 W1, b1, W2, b2, Wc2, bc2,
                 h1_ref, catL_ref, catR_ref, hn2_ref, cen2_ref):
    t = cen_ref[...] + p0_ref[...] + p1_ref[...]
    t = _ln(t, g[...], b[...])
    h1 = h_ref[...] + _dot(jnp.maximum(t, 0.0), Wo[...]) + bo[...]
    h1_ref[...] = h1
    catL_ref[:, 0:256] = _dot(h1, WnlL[...])
    catL_ref[:, 256:384] = _dot(h1, WnfL[...]) + bnfL[...]
    catR_ref[:, 0:256] = _dot(h1, WnlR[...])
    catR_ref[:, 256:384] = _dot(h1, WnfR[...]) + bnfR[...]
    u = jnp.maximum(_dot(h1, W1[...]) + b1[...], 0.0)
    hn2_ref[...] = _dot(u, W2[...]) + b2[...]
    cen2_ref[...] = _dot(h1, Wc2[...]) + bc2[...]


def _ebond_body(hb_ref, gl_ref, gr_ref,
                WblL, Wl1, bl1, Wl2, bl2, WblR, Wr1, br1, Wr2, br2,
                ml_ref, mr_ref):
    hb = hb_ref[...]
    gl = gl_ref[...]
    gr = gr_ref[...]
    il = _dot(hb, WblL[...]) * gl[:, 0:256]
    tl = jnp.maximum(_dot(il, Wl1[...]) + bl1[...], 0.0)
    ml_ref[...] = _dot(tl, Wl2[...]) + bl2[...]
    ir = _dot(hb, WblR[...]) * gr[:, 0:256]
    tr = jnp.maximum(_dot(ir, Wr1[...]) + br1[...], 0.0)
    mr_ref[...] = _dot(tr, Wr2[...]) + br2[...]


def _combine_body(a0_ref, a1_ref, b0_ref, b1_ref, a_ref, b_ref):
    a_ref[...] = a0_ref[...] + a1_ref[...]
    b_ref[...] = b0_ref[...] + b1_ref[...]


def _bfin_body(hb_ref, al_ref, ar_ref, gl_ref, gr_ref, g2_ref,
               Ws, bs, g, b, Wo, bo, We1, be1, We2, be2, Wm2, bm2,
               hb1_ref, msg2_ref):
    hb = hb_ref[...]
    s = (al_ref[...] + ar_ref[...] + gl_ref[:, 256:384] + gr_ref[:, 256:384]
         + _dot(hb, Ws[...]) + bs[...])
    s = _ln(s, g[...], b[...])
    hb1 = hb + _dot(jnp.maximum(s, 0.0), Wo[...]) + bo[...]
    hb1_ref[...] = hb1
    t = jnp.maximum(_dot(hb1, We1[...]) + be1[...], 0.0)
    he = _dot(t, We2[...]) + be2[...]
    msg2_ref[...] = _dot(he * g2_ref[...], Wm2[...]) + bm2[...]


def _node_c_body(h1_ref, cen2_ref, q0_ref, q1_ref, Wo, bo, g, b, h2_ref):
    t = cen2_ref[...] + q0_ref[...] + q1_ref[...]
    t = _ln(t, g[...], b[...])
    h2_ref[...] = h1_ref[...] + _dot(jnp.maximum(t, 0.0), Wo[...]) + bo[...]


def kernel(h_node, pos_node, h_bond, bond_index, batch, is_mol, is_frag, params):
    p = params
    nbe, bb, nbb = p["nbe"][0], p["bb"][0], p["nbb"][0]
    row = bond_index[0]
    col = bond_index[1]

    pos16 = jnp.zeros((N_NODES, 16), F32).at[:, 0:3].set(pos_node)
    off = _r2(jnp.linspace(0.0, CUTOFF, NUM_GAUSS).astype(F32))

    h, hn1, cen1 = _tc_call(
        _node_a_body, N_NODES, T_N,
        [(h_node, True),
         (p["lin_node"]["W"], False), (_r2(p["lin_node"]["b"]), False),
         (nbe["node_net"]["l1"]["W"], False), (_r2(nbe["node_net"]["l1"]["b"]), False),
         (nbe["node_net"]["l2"]["W"], False), (_r2(nbe["node_net"]["l2"]["b"]), False),
         (nbe["centroid"]["W"], False), (_r2(nbe["centroid"]["b"]), False)],
        [D, D, D])

    pr, pc = _multi_gather([(pos16, row), (pos16, col)], tc_tiling=False)
    (g1,) = _multi_gather([(hn1, col)])

    (msg,) = _tc_call(
        _edge1_body, N_EDGES, T_E,
        [(pr, True), (pc, True), (g1, True), (off, False),
         (p["edge_emb"]["W"], False), (_r2(p["edge_emb"]["b"]), False),
         (nbe["edge_net"]["l1"]["W"], False), (_r2(nbe["edge_net"]["l1"]["b"]), False),
         (nbe["edge_net"]["l2"]["W"], False), (_r2(nbe["edge_net"]["l2"]["b"]), False),
         (nbe["msg_net"]["W"], False), (_r2(nbe["msg_net"]["b"]), False)],
        [D])

    aggr1p = _scatter_add(msg, row)

    h1, catL, catR, hn2, cen2 = _tc_call(
        _node_b_body, N_NODES, T_N,
        [(h, True), (aggr1p[0], True), (aggr1p[1], True), (cen1, True),
         (nbe["out"]["W"], False), (_r2(nbe["out"]["b"]), False),
         (_r2(nbe["ln_g"]), False), (_r2(nbe["ln_b"]), False),
         (bb["ffn_l"]["node_lin"]["W"], False),
         (bb["node_ffn_l"]["W"], False), (_r2(bb["node_ffn_l"]["b"]), False),
         (bb["ffn_r"]["node_lin"]["W"], False),
         (bb["node_ffn_r"]["W"], False), (_r2(bb["node_ffn_r"]["b"]), False),
         (nbb["node_net"]["l1"]["W"], False), (_r2(nbb["node_net"]["l1"]["b"]), False),
         (nbb["node_net"]["l2"]["W"], False), (_r2(nbb["node_net"]["l2"]["b"]), False),
         (nbb["centroid"]["W"], False), (_r2(nbb["centroid"]["b"]), False)],
        [D, 384, 384, D, D])

    gl, gr, g2 = _multi_gather([(catL, row), (catR, col), (hn2, col)])

    m_l, m_r = _tc_call(
        _ebond_body, N_EDGES, T_E,
        [(h_bond, True), (gl, True), (gr, True),
         (bb["ffn_l"]["bond_lin"]["W"], False),
         (bb["ffn_l"]["inter"]["l1"]["W"], False), (_r2(bb["ffn_l"]["inter"]["l1"]["b"]), False),
         (bb["ffn_l"]["inter"]["l2"]["W"], False), (_r2(bb["ffn_l"]["inter"]["l2"]["b"]), False),
         (bb["ffn_r"]["bond_lin"]["W"], False),
         (bb["ffn_r"]["inter"]["l1"]["W"], False), (_r2(bb["ffn_r"]["inter"]["l1"]["b"]), False),
         (bb["ffn_r"]["inter"]["l2"]["W"], False), (_r2(bb["ffn_r"]["inter"]["l2"]["b"]), False)],
        [D, D])

    aggLp = _scatter_add(m_l, col)
    aggRp = _scatter_add(m_r, row)

    aggL, aggR = _tc_call(
        _combine_body, N_NODES, T_N,
        [(aggLp[0], True), (aggLp[1], True), (aggRp[0], True), (aggRp[1], True)],
        [D, D])

    al, ar = _multi_gather([(aggL, row), (aggR, col)])

    hb1, msg2 = _tc_call(
        _bfin_body, N_EDGES, T_E,
        [(h_bond, True), (al, True), (ar, True), (gl, True), (gr, True), (g2, True),
         (bb["self_ffn"]["W"], False), (_r2(bb["self_ffn"]["b"]), False),
         (_r2(bb["ln_g"]), False), (_r2(bb["ln_b"]), False),
         (bb["out"]["W"], False), (_r2(bb["out"]["b"]), False),
         (nbb["edge_net"]["l1"]["W"], False), (_r2(nbb["edge_net"]["l1"]["b"]), False),
         (nbb["edge_net"]["l2"]["W"], False), (_r2(nbb["edge_net"]["l2"]["b"]), False),
         (nbb["msg_net"]["W"], False), (_r2(nbb["msg_net"]["b"]), False)],
        [D, D])

    aggr2p = _scatter_add(msg2, row)

    (h2,) = _tc_call(
        _node_c_body, N_NODES, T_N,
        [(h1, True), (cen2, True), (aggr2p[0], True), (aggr2p[1], True),
         (nbb["out"]["W"], False), (_r2(nbb["out"]["b"]), False),
         (_r2(nbb["ln_g"]), False), (_r2(nbb["ln_b"]), False)],
        [D])

    return h2, hb1
